# baseline (speedup 1.0000x reference)
import jax
import jax.numpy as jnp
from jax import lax
from jax.experimental import pallas as pl
from jax.experimental.pallas import tpu as pltpu
from jax.experimental.pallas import tpu_sc as plsc

N = 10000
D = 128
NC = 2
NS = 16
NW = NC * NS
CHUNK = 128
NCH = 80
G = 8
NG = NCH // G
EPT = NCH * CHUNK
E_PAD = NW * EPT
N_PAD = 10240
ROWS_PER_TILE = N_PAD // NS
ROW_BYTES = CHUNK * D * 4


def _sc_agg_body(pk_hbm, ev_hbm, x_hbm, part_hbm, pk_v, ev_v, rows, acc,
                 gsem, psem, ssem):
    c = lax.axis_index("c")
    s = lax.axis_index("s")
    wid = s * NC + c

    zero16 = jnp.zeros((16,), jnp.float32)

    def _zero_row(i, _):
        for k8 in range(8):
            rows[0, i, pl.ds(k8 * 16, 16)] = zero16
        return 0

    lax.fori_loop(0, CHUNK, _zero_row, 0)

    base = s * ROWS_PER_TILE
    for k in range(5):
        pltpu.sync_copy(rows.at[0], acc.at[pl.ds(base + k * 128, 128)])

    pltpu.async_copy(pk_hbm.at[wid].at[pl.ds(0, G)], pk_v.at[0], psem)
    pltpu.async_copy(ev_hbm.at[wid].at[pl.ds(0, G)], ev_v.at[0], psem)

    plsc.subcore_barrier()

    def _group(g2, gg, _):
        g = g2 * 2 + gg
        pltpu.make_async_copy(pk_hbm.at[0].at[pl.ds(0, G)], pk_v.at[gg],
                              psem).wait()
        pltpu.make_async_copy(ev_hbm.at[0].at[pl.ds(0, G)], ev_v.at[gg],
                              psem).wait()

        @pl.when(g > 0)
        def _():
            for _b in range(2):
                pltpu.make_async_copy(x_hbm.at[pl.ds(0, CHUNK)],
                                      rows.at[_b], ssem).wait()

        @pl.when(g + 1 < NG)
        def _():
            pltpu.async_copy(pk_hbm.at[wid].at[pl.ds((g + 1) * G, G)],
                             pk_v.at[1 - gg], psem)
            pltpu.async_copy(ev_hbm.at[wid].at[pl.ds((g + 1) * G, G)],
                             ev_v.at[1 - gg], psem)

        def _pair(p, _):
            @pl.when(p > 0)
            def _():
                for _b in range(2):
                    pltpu.make_async_copy(x_hbm.at[pl.ds(0, CHUNK)],
                                          rows.at[_b], ssem).wait()

            cps = [
                pltpu.async_copy(
                    x_hbm.at[pk_v.at[gg, p * 2 + b, 0]], rows.at[b], gsem)
                for b in range(2)
            ]
            for b in range(2):
                cc = p * 2 + b
                cps[b].wait()

                def _scale(i16, _):
                    evs = ev_v[gg, cc, pl.ds(i16 * 16, 16)]
                    for t in range(16):
                        evv = evs.at[jnp.full((16,), t, jnp.int32)].get(
                            mode="promise_in_bounds")
                        i = i16 * 16 + t
                        for k8 in range(8):
                            sl = pl.ds(k8 * 16, 16)
                            rows[b, i, sl] = rows[b, i, sl] * evv
                    return 0

                lax.fori_loop(0, CHUNK // 16, _scale, 0)

                pltpu.async_copy(rows.at[b], acc.at[pk_v.at[gg, cc, 1]], ssem,
                                 add=True)
            return 0

        lax.fori_loop(0, G // 2, _pair, 0)
        return 0

    lax.fori_loop(0, NG // 2,
                  lambda g2, _: (_group(g2, 0, _), _group(g2, 1, _))[1], 0)

    for _b in range(2):
        pltpu.make_async_copy(x_hbm.at[pl.ds(0, CHUNK)], rows.at[_b],
                              ssem).wait()

    plsc.subcore_barrier()

    pltpu.sync_copy(acc.at[pl.ds(base, ROWS_PER_TILE)],
                    part_hbm.at[c].at[pl.ds(base, ROWS_PER_TILE)])


_sc_agg = pl.kernel(
    _sc_agg_body,
    out_type=jax.ShapeDtypeStruct((NC, N_PAD, D), jnp.float32),
    mesh=plsc.VectorSubcoreMesh(core_axis_name="c", subcore_axis_name="s"),
    scratch_types=[
        pltpu.VMEM((2, G, 2, CHUNK), jnp.int32),
        pltpu.VMEM((2, G, CHUNK), jnp.float32),
        pltpu.VMEM((2, CHUNK, D), jnp.float32),
        pltpu.VMEM_SHARED((N_PAD, D), jnp.float32),
        pltpu.SemaphoreType.DMA,
        pltpu.SemaphoreType.DMA,
        pltpu.SemaphoreType.DMA,
    ],
)


def _tc_linear_body(p_ref, w_ref, b_ref, o_ref):
    sblk = p_ref[0] + p_ref[1]
    acc = lax.dot_general(sblk, w_ref[...], (((1,), (1,)), ((), ())),
                          preferred_element_type=jnp.float32)
    o_ref[...] = acc + b_ref[...]


BLK = 1000

_tc_linear = pl.pallas_call(
    _tc_linear_body,
    grid=(N // BLK,),
    in_specs=[
        pl.BlockSpec((NC, BLK, D), lambda i: (0, i, 0)),
        pl.BlockSpec((D, D), lambda i: (0, 0)),
        pl.BlockSpec((1, D), lambda i: (0, 0)),
    ],
    out_specs=pl.BlockSpec((BLK, D), lambda i: (i, 0)),
    out_shape=jax.ShapeDtypeStruct((N, D), jnp.float32),
)


@jax.jit
def kernel(edge_index, edge_vals, x, W, b):
    E = edge_vals.shape[0]
    pad = E_PAD - E
    src = jnp.concatenate([edge_index[1], jnp.zeros((pad,), jnp.int32)])
    dst = jnp.concatenate([edge_index[0], jnp.zeros((pad,), jnp.int32)])
    ev = jnp.concatenate([edge_vals, jnp.zeros((pad,), jnp.float32)])
    pk = jnp.stack([src.reshape(NW, NCH, CHUNK), dst.reshape(NW, NCH, CHUNK)],
                   axis=2)
    parts = _sc_agg(pk, ev.reshape(NW, NCH, CHUNK), x)
    return _tc_linear(parts, W, b.reshape(1, D))


# speedup vs baseline: 1.1011x; 1.1011x over previous
import jax
import jax.numpy as jnp
from jax import lax
from jax.experimental import pallas as pl
from jax.experimental.pallas import tpu as pltpu
from jax.experimental.pallas import tpu_sc as plsc

N = 10000
D = 128
NC = 2
NS = 16
NW = NC * NS
CHUNK = 128
G = 8
NCH0 = 112
NCH1 = 48
TOTAL_CH = NS * (NCH0 + NCH1)
E_PAD = TOTAL_CH * CHUNK
N_PAD = 10240
ROWS_PER_TILE = N_PAD // NS
ROW_BYTES = CHUNK * D * 4


def _sc_agg_body(pk_hbm, ev_hbm, x_hbm, part_hbm, pk_v, ev_v, rows, acc,
                 gsem, psem, ssem):
    c = lax.axis_index("c")
    s = lax.axis_index("s")
    n_ch = jnp.where(c == 0, NCH0, NCH1)
    ng = n_ch // G
    base_ch = jnp.where(c == 0, s * NCH0, NS * NCH0 + s * NCH1)

    zero16 = jnp.zeros((16,), jnp.float32)

    def _zero_row(i, _):
        for k8 in range(8):
            rows[0, i, pl.ds(k8 * 16, 16)] = zero16
        return 0

    lax.fori_loop(0, CHUNK, _zero_row, 0)

    base = s * ROWS_PER_TILE
    for k in range(5):
        pltpu.sync_copy(rows.at[0], acc.at[pl.ds(base + k * 128, 128)])

    pltpu.async_copy(pk_hbm.at[pl.ds(base_ch, G)], pk_v.at[0], psem)
    pltpu.async_copy(ev_hbm.at[pl.ds(base_ch, G)], ev_v.at[0], psem)

    plsc.subcore_barrier()

    def _group(g2, gg, _):
        g = g2 * 2 + gg
        pltpu.make_async_copy(pk_hbm.at[pl.ds(0, G)], pk_v.at[gg],
                              psem).wait()
        pltpu.make_async_copy(ev_hbm.at[pl.ds(0, G)], ev_v.at[gg],
                              psem).wait()

        @pl.when(g > 0)
        def _():
            for _b in range(2):
                pltpu.make_async_copy(x_hbm.at[pl.ds(0, CHUNK)],
                                      rows.at[_b], ssem).wait()

        @pl.when(g + 1 < ng)
        def _():
            pltpu.async_copy(pk_hbm.at[pl.ds(base_ch + (g + 1) * G, G)],
                             pk_v.at[1 - gg], psem)
            pltpu.async_copy(ev_hbm.at[pl.ds(base_ch + (g + 1) * G, G)],
                             ev_v.at[1 - gg], psem)

        def _pair(p, _):
            @pl.when(p > 0)
            def _():
                for _b in range(2):
                    pltpu.make_async_copy(x_hbm.at[pl.ds(0, CHUNK)],
                                          rows.at[_b], ssem).wait()

            cps = [
                pltpu.async_copy(
                    x_hbm.at[pk_v.at[gg, p * 2 + b, 0]], rows.at[b], gsem)
                for b in range(2)
            ]
            for b in range(2):
                cc = p * 2 + b
                cps[b].wait()

                def _scale(i16, _):
                    evs = ev_v[gg, cc, pl.ds(i16 * 16, 16)]
                    for t in range(16):
                        evv = evs.at[jnp.full((16,), t, jnp.int32)].get(
                            mode="promise_in_bounds")
                        i = i16 * 16 + t
                        for k8 in range(8):
                            sl = pl.ds(k8 * 16, 16)
                            rows[b, i, sl] = rows[b, i, sl] * evv
                    return 0

                lax.fori_loop(0, CHUNK // 16, _scale, 0)

                pltpu.async_copy(rows.at[b], acc.at[pk_v.at[gg, cc, 1]], ssem,
                                 add=True)
            return 0

        lax.fori_loop(0, G // 2, _pair, 0)
        return 0

    lax.fori_loop(0, ng // 2,
                  lambda g2, _: (_group(g2, 0, _), _group(g2, 1, _))[1], 0)

    for _b in range(2):
        pltpu.make_async_copy(x_hbm.at[pl.ds(0, CHUNK)], rows.at[_b],
                              ssem).wait()

    plsc.subcore_barrier()

    pltpu.sync_copy(acc.at[pl.ds(base, ROWS_PER_TILE)],
                    part_hbm.at[c].at[pl.ds(base, ROWS_PER_TILE)])


_sc_agg = pl.kernel(
    _sc_agg_body,
    out_type=jax.ShapeDtypeStruct((NC, N_PAD, D), jnp.float32),
    mesh=plsc.VectorSubcoreMesh(core_axis_name="c", subcore_axis_name="s"),
    scratch_types=[
        pltpu.VMEM((2, G, 2, CHUNK), jnp.int32),
        pltpu.VMEM((2, G, CHUNK), jnp.float32),
        pltpu.VMEM((2, CHUNK, D), jnp.float32),
        pltpu.VMEM_SHARED((N_PAD, D), jnp.float32),
        pltpu.SemaphoreType.DMA,
        pltpu.SemaphoreType.DMA,
        pltpu.SemaphoreType.DMA,
    ],
)


def _tc_linear_body(p_ref, w_ref, b_ref, o_ref):
    sblk = p_ref[0] + p_ref[1]
    acc = lax.dot_general(sblk, w_ref[...], (((1,), (1,)), ((), ())),
                          preferred_element_type=jnp.float32)
    o_ref[...] = acc + b_ref[...]


BLK = 1000

_tc_linear = pl.pallas_call(
    _tc_linear_body,
    grid=(N // BLK,),
    in_specs=[
        pl.BlockSpec((NC, BLK, D), lambda i: (0, i, 0)),
        pl.BlockSpec((D, D), lambda i: (0, 0)),
        pl.BlockSpec((1, D), lambda i: (0, 0)),
    ],
    out_specs=pl.BlockSpec((BLK, D), lambda i: (i, 0)),
    out_shape=jax.ShapeDtypeStruct((N, D), jnp.float32),
)


@jax.jit
def kernel(edge_index, edge_vals, x, W, b):
    E = edge_vals.shape[0]
    pad = E_PAD - E
    src = jnp.concatenate([edge_index[1], jnp.zeros((pad,), jnp.int32)])
    dst = jnp.concatenate([edge_index[0], jnp.zeros((pad,), jnp.int32)])
    ev = jnp.concatenate([edge_vals, jnp.zeros((pad,), jnp.float32)])
    pk = jnp.stack([src.reshape(TOTAL_CH, CHUNK), dst.reshape(TOTAL_CH, CHUNK)],
                   axis=1)
    parts = _sc_agg(pk, ev.reshape(TOTAL_CH, CHUNK), x)
    return _tc_linear(parts, W, b.reshape(1, D))


# speedup vs baseline: 2.3380x; 2.1233x over previous
import jax
import jax.numpy as jnp
from jax import lax
from jax.experimental import pallas as pl
from jax.experimental.pallas import tpu as pltpu
from jax.experimental.pallas import tpu_sc as plsc

N = 10000
D = 128
NC = 2
NS = 16
NW = NC * NS
CHUNK = 128
G = 8
NCH0 = 80
NCH1 = 80
TOTAL_CH = NS * (NCH0 + NCH1)
E_PAD = TOTAL_CH * CHUNK
N_PAD = 10240
ROWS_PER_TILE = N_PAD // NS
ROW_BYTES = CHUNK * D * 4


def _sc_agg_body(pk_hbm, ev_hbm, x_hbm, part_hbm, pk_v, ev_v, rows, acc,
                 gsem, psem, ssem):
    c = lax.axis_index("c")
    s = lax.axis_index("s")
    n_ch = jnp.where(c == 0, NCH0, NCH1)
    ng = n_ch // G
    base_ch = jnp.where(c == 0, s * NCH0, NS * NCH0 + s * NCH1)

    zero16 = jnp.zeros((16,), jnp.float32)

    def _zero_row(i, _):
        for k8 in range(8):
            rows[0, i, pl.ds(k8 * 16, 16)] = zero16
        return 0

    lax.fori_loop(0, CHUNK, _zero_row, 0)

    base = s * ROWS_PER_TILE
    for k in range(5):
        pltpu.sync_copy(rows.at[0], acc.at[pl.ds(base + k * 128, 128)])

    pltpu.async_copy(pk_hbm.at[pl.ds(base_ch, G)], pk_v.at[0], psem)
    pltpu.async_copy(ev_hbm.at[pl.ds(base_ch, G)], ev_v.at[0], psem)

    plsc.subcore_barrier()

    def _group(g2, gg, _):
        g = g2 * 2 + gg
        pltpu.make_async_copy(pk_hbm.at[pl.ds(0, G)], pk_v.at[gg],
                              psem).wait()
        pltpu.make_async_copy(ev_hbm.at[pl.ds(0, G)], ev_v.at[gg],
                              psem).wait()

        @pl.when(g > 0)
        def _():
            for _b in range(2):
                pltpu.make_async_copy(x_hbm.at[pl.ds(0, CHUNK)],
                                      rows.at[_b], ssem).wait()

        @pl.when(g + 1 < ng)
        def _():
            pltpu.async_copy(pk_hbm.at[pl.ds(base_ch + (g + 1) * G, G)],
                             pk_v.at[1 - gg], psem)
            pltpu.async_copy(ev_hbm.at[pl.ds(base_ch + (g + 1) * G, G)],
                             ev_v.at[1 - gg], psem)

        def _pair(p, _):
            @pl.when(p > 0)
            def _():
                for _b in range(2):
                    pltpu.make_async_copy(x_hbm.at[pl.ds(0, CHUNK)],
                                          rows.at[_b], ssem).wait()

            cps = [
                pltpu.async_copy(
                    x_hbm.at[pk_v.at[gg, p * 2 + b, 0]], rows.at[b], gsem)
                for b in range(2)
            ]
            for b in range(2):
                cc = p * 2 + b
                cps[b].wait()

                def _scale(i16, _):
                    evs = ev_v[gg, cc, pl.ds(i16 * 16, 16)]
                    for t in range(16):
                        evv = evs.at[jnp.full((16,), t, jnp.int32)].get(
                            mode="promise_in_bounds")
                        i = i16 * 16 + t
                        for k8 in range(8):
                            sl = pl.ds(k8 * 16, 16)
                            rows[b, i, sl] = rows[b, i, sl] * evv
                    return 0

                lax.fori_loop(0, CHUNK // 16, _scale, 0)

                pltpu.async_copy(rows.at[b], acc.at[pk_v.at[gg, cc, 1]], ssem,
                                 add=True)
            return 0

        lax.fori_loop(0, G // 2, _pair, 0)
        return 0

    lax.fori_loop(0, ng // 2,
                  lambda g2, _: (_group(g2, 0, _), _group(g2, 1, _))[1], 0)

    for _b in range(2):
        pltpu.make_async_copy(x_hbm.at[pl.ds(0, CHUNK)], rows.at[_b],
                              ssem).wait()

    plsc.subcore_barrier()

    pltpu.sync_copy(acc.at[pl.ds(base, ROWS_PER_TILE)],
                    part_hbm.at[c].at[pl.ds(base, ROWS_PER_TILE)])


_sc_agg = pl.kernel(
    _sc_agg_body,
    out_type=jax.ShapeDtypeStruct((NC, N_PAD, D), jnp.float32),
    mesh=plsc.VectorSubcoreMesh(core_axis_name="c", subcore_axis_name="s"),
    scratch_types=[
        pltpu.VMEM((2, G, 2, CHUNK), jnp.int32),
        pltpu.VMEM((2, G, CHUNK), jnp.float32),
        pltpu.VMEM((2, CHUNK, D), jnp.float32),
        pltpu.VMEM_SHARED((N_PAD, D), jnp.float32),
        pltpu.SemaphoreType.DMA,
        pltpu.SemaphoreType.DMA,
        pltpu.SemaphoreType.DMA,
    ],
)


def _tc_linear_body(p_ref, w_ref, b_ref, o_ref):
    sblk = p_ref[0] + p_ref[1]
    acc = lax.dot_general(sblk, w_ref[...], (((1,), (1,)), ((), ())),
                          preferred_element_type=jnp.float32)
    o_ref[...] = acc + b_ref[...]


BLK = 1000

_tc_linear = pl.pallas_call(
    _tc_linear_body,
    grid=(N // BLK,),
    in_specs=[
        pl.BlockSpec((NC, BLK, D), lambda i: (0, i, 0)),
        pl.BlockSpec((D, D), lambda i: (0, 0)),
        pl.BlockSpec((1, D), lambda i: (0, 0)),
    ],
    out_specs=pl.BlockSpec((BLK, D), lambda i: (i, 0)),
    out_shape=jax.ShapeDtypeStruct((N, D), jnp.float32),
)


@jax.jit
def kernel(edge_index, edge_vals, x, W, b):
    E = edge_vals.shape[0]
    pad = E_PAD - E
    spread = jnp.arange(pad, dtype=jnp.int32) % N
    src = jnp.concatenate([edge_index[1], spread])
    dst = jnp.concatenate([edge_index[0], spread])
    ev = jnp.concatenate([edge_vals, jnp.zeros((pad,), jnp.float32)])
    pk = jnp.stack([src.reshape(TOTAL_CH, CHUNK), dst.reshape(TOTAL_CH, CHUNK)],
                   axis=1)
    parts = _sc_agg(pk, ev.reshape(TOTAL_CH, CHUNK), x)
    return _tc_linear(parts, W, b.reshape(1, D))
